# baseline (speedup 1.0000x reference)
import functools

import jax
import jax.numpy as jnp
import numpy as np
from jax import lax
from jax.experimental import pallas as pl
from jax.experimental.pallas import tpu as pltpu
from jax.experimental.pallas import tpu_sc as plsc

_B = 16384
_L = 200
_V = 21
_E = 128

_SPLIT = 8192

_NC = 2
_NS = 16
_NW = _NC * _NS
_RW = _SPLIT // _NW
_STRIDE = 128
_CR = 128
_NCHUNK = _RW // _CR
_CW = _CR * _L
_GPC = _CR // 16
_UNROLL = 8


def _pe_mean() -> np.ndarray:
    pos = np.arange(_L, dtype=np.float32)
    ang = (2.0 * np.pi * pos / float(_L)).astype(np.float32)
    freqs = np.arange(1, _E // 2 + 1, dtype=np.float32)
    phase = ang[:, None] * freqs[None, :]
    pe = np.concatenate([np.sin(phase), np.cos(phase)], axis=-1)
    return pe.mean(axis=0).astype(np.float32)


_PE_MEAN = _pe_mean()

_sc_mesh = plsc.VectorSubcoreMesh(
    core_axis_name="c", subcore_axis_name="s",
    num_cores=_NC, num_subcores=_NS)


@functools.partial(
    pl.kernel,
    out_type=jax.ShapeDtypeStruct((_SPLIT * _STRIDE,), jnp.float32),
    mesh=_sc_mesh,
    scratch_types=[
        pltpu.VMEM((_CW,), jnp.int32),
        pltpu.VMEM((_CW,), jnp.int32),
        pltpu.VMEM((_RW * _STRIDE,), jnp.float32),
        pltpu.SemaphoreType.DMA,
        pltpu.SemaphoreType.DMA,
    ],
    compiler_params=pltpu.CompilerParams(needs_layout_passes=False),
)
def _sc_hist(traj_hbm, counts_hbm, bufa, bufb, counts_v, sema, semb):
    c = lax.axis_index("c")
    s = lax.axis_index("s")
    wid = s * _NC + c
    row0 = wid * _RW
    lane = lax.iota(jnp.int32, 16)
    lbase = lane * _L
    ones = jnp.full((16,), 1.0, jnp.float32)
    zeros = jnp.zeros((16,), jnp.float32)

    bufs = (bufa, bufb)
    sems = (sema, semb)

    def chunk_src(ci):
        base = (row0 + ci * _CR) * _L
        return traj_hbm.at[pl.ds(base, _CW)]

    pltpu.async_copy(chunk_src(0), bufs[0], sems[0])

    @plsc.parallel_loop(0, _RW * _STRIDE, step=16, unroll=8)
    def _zero(i):
        counts_v[pl.ds(i, 16)] = zeros

    for ci in range(_NCHUNK):
        buf = bufs[ci % 2]
        pltpu.make_async_copy(chunk_src(ci), buf, sems[ci % 2]).wait()
        if ci + 1 < _NCHUNK:
            pltpu.async_copy(chunk_src(ci + 1), bufs[(ci + 1) % 2],
                             sems[(ci + 1) % 2])
        for g in range(_GPC):
            rowbase = (ci * _CR + g * 16 + lane) * _STRIDE
            gbase = g * 16 * _L + lbase

            @plsc.parallel_loop(0, _L, unroll=_UNROLL, carry=gbase)
            def _hist(l, idxv):
                tok = plsc.load_gather(buf, [idxv])
                plsc.addupdate_scatter(counts_v, [rowbase + tok], ones)
                return idxv + 1

    pltpu.sync_copy(counts_v,
                    counts_hbm.at[pl.ds(row0 * _STRIDE, _RW * _STRIDE)])


_BR1 = 512
_BTC = _B - _SPLIT


def _tc_hist_mm_body(tr_ref, tab_ref, pe_ref, out_ref):
    t = tr_ref[...]
    cols = []
    for v in range(_V):
        m = (t == v).astype(jnp.float32)
        cols.append(jnp.sum(m, axis=1, keepdims=True))
    counts = jnp.concatenate(cols, axis=1)
    acc = lax.dot_general(
        counts, tab_ref[...], (((1,), (0,)), ((), ())),
        preferred_element_type=jnp.float32)
    out_ref[...] = acc * (1.0 / _L) + pe_ref[0:1, :]


_BR2 = 1024


def _mm_body(cnt_ref, tab_ref, pe_ref, outbuf_ref, out_ref):
    acc = lax.dot_general(
        cnt_ref[...], tab_ref[...], (((1,), (0,)), ((), ())),
        preferred_element_type=jnp.float32)
    out_ref[...] = acc * (1.0 / _L) + pe_ref[0:1, :]


@jax.jit
def kernel(trajs, table):
    trajs = trajs.astype(jnp.int32)
    pe = jnp.broadcast_to(jnp.asarray(_PE_MEAN)[None, :], (8, _E))

    traj_flat = trajs[_BTC:].reshape(-1)
    counts = _sc_hist(traj_flat).reshape(_SPLIT, _STRIDE)

    out_tc = pl.pallas_call(
        _tc_hist_mm_body,
        grid=(_BTC // _BR1,),
        in_specs=[
            pl.BlockSpec((_BR1, _L), lambda i: (i, 0)),
            pl.BlockSpec((_V, _E), lambda i: (0, 0)),
            pl.BlockSpec((8, _E), lambda i: (0, 0)),
        ],
        out_specs=pl.BlockSpec((_BR1, _E), lambda i: (i, 0)),
        out_shape=jax.ShapeDtypeStruct((_B, _E), jnp.float32),
    )(trajs, table, pe)

    tab_pad = jnp.zeros((_STRIDE, _E), jnp.float32).at[:_V].set(table)
    return pl.pallas_call(
        _mm_body,
        grid=(_SPLIT // _BR2,),
        in_specs=[
            pl.BlockSpec((_BR2, _STRIDE), lambda i: (i, 0)),
            pl.BlockSpec((_STRIDE, _E), lambda i: (0, 0)),
            pl.BlockSpec((8, _E), lambda i: (0, 0)),
            pl.BlockSpec(memory_space=pl.ANY),
        ],
        out_specs=pl.BlockSpec((_BR2, _E), lambda i: (i + _BTC // _BR2, 0)),
        out_shape=jax.ShapeDtypeStruct((_B, _E), jnp.float32),
        input_output_aliases={3: 0},
    )(counts, tab_pad, pe, out_tc)


# speedup vs baseline: 1.3914x; 1.3914x over previous
import functools

import jax
import jax.numpy as jnp
import numpy as np
from jax import lax
from jax.experimental import pallas as pl
from jax.experimental.pallas import tpu as pltpu
from jax.experimental.pallas import tpu_sc as plsc

_B = 16384
_L = 200
_V = 21
_E = 128

_SPLIT = 8192

_NC = 2
_NS = 16
_NW = _NC * _NS
_RW = _SPLIT // _NW
_STRIDE = 128
_CR = 128
_NCHUNK = _RW // _CR
_CW = _CR * _L
_GPC = _CR // 16
_UNROLL = 8


def _pe_mean() -> np.ndarray:
    pos = np.arange(_L, dtype=np.float32)
    ang = (2.0 * np.pi * pos / float(_L)).astype(np.float32)
    freqs = np.arange(1, _E // 2 + 1, dtype=np.float32)
    phase = ang[:, None] * freqs[None, :]
    pe = np.concatenate([np.sin(phase), np.cos(phase)], axis=-1)
    return pe.mean(axis=0).astype(np.float32)


_PE_MEAN = _pe_mean()

_sc_mesh = plsc.VectorSubcoreMesh(
    core_axis_name="c", subcore_axis_name="s",
    num_cores=_NC, num_subcores=_NS)


@functools.partial(
    pl.kernel,
    out_type=jax.ShapeDtypeStruct((_SPLIT * _STRIDE,), jnp.float32),
    mesh=_sc_mesh,
    scratch_types=[
        pltpu.VMEM((_CR, _L), jnp.int32),
        pltpu.VMEM((_CR, _L), jnp.int32),
        pltpu.VMEM((_RW * _STRIDE,), jnp.float32),
        pltpu.SemaphoreType.DMA,
        pltpu.SemaphoreType.DMA,
    ],
    compiler_params=pltpu.CompilerParams(needs_layout_passes=False, use_tc_tiling_on_sc=True),
)
def _sc_hist(traj_hbm, counts_hbm, bufa, bufb, counts_v, sema, semb):
    c = lax.axis_index("c")
    s = lax.axis_index("s")
    wid = s * _NC + c
    row0 = _BTC + wid * _RW
    lane = lax.iota(jnp.int32, 16)
    ones = jnp.full((16,), 1.0, jnp.float32)
    zeros = jnp.zeros((16,), jnp.float32)

    bufs = (bufa, bufb)
    sems = (sema, semb)

    def chunk_src(ci):
        return traj_hbm.at[pl.ds(row0 + ci * _CR, _CR), :]

    pltpu.async_copy(chunk_src(0), bufs[0], sems[0])

    @plsc.parallel_loop(0, _RW * _STRIDE, step=16, unroll=8)
    def _zero(i):
        counts_v[pl.ds(i, 16)] = zeros

    for ci in range(_NCHUNK):
        buf = bufs[ci % 2]
        pltpu.make_async_copy(chunk_src(ci), buf, sems[ci % 2]).wait()
        if ci + 1 < _NCHUNK:
            pltpu.async_copy(chunk_src(ci + 1), bufs[(ci + 1) % 2],
                             sems[(ci + 1) % 2])
        for g in range(_GPC):
            rowbase = (ci * _CR + g * 16 + lane) * _STRIDE
            rowv = g * 16 + lane
            col0 = jnp.zeros((16,), jnp.int32)

            @plsc.parallel_loop(0, _L, unroll=_UNROLL, carry=col0)
            def _hist(l, colv):
                tok = plsc.load_gather(buf, [rowv, colv])
                plsc.addupdate_scatter(counts_v, [rowbase + tok], ones)
                return colv + 1

    pltpu.sync_copy(
        counts_v,
        counts_hbm.at[pl.ds((row0 - _BTC) * _STRIDE, _RW * _STRIDE)])


_BR1 = 512
_BTC = _B - _SPLIT


def _tc_hist_mm_body(tr_ref, tab_ref, pe_ref, out_ref):
    t = tr_ref[...]
    cols = []
    for v in range(_V):
        m = (t == v).astype(jnp.float32)
        cols.append(jnp.sum(m, axis=1, keepdims=True))
    counts = jnp.concatenate(cols, axis=1)
    acc = lax.dot_general(
        counts, tab_ref[...], (((1,), (0,)), ((), ())),
        preferred_element_type=jnp.float32)
    out_ref[...] = acc * (1.0 / _L) + pe_ref[0:1, :]


_BR2 = 1024


def _mm_body(cnt_ref, tab_ref, pe_ref, outbuf_ref, out_ref):
    acc = lax.dot_general(
        cnt_ref[...], tab_ref[...], (((1,), (0,)), ((), ())),
        preferred_element_type=jnp.float32)
    out_ref[...] = acc * (1.0 / _L) + pe_ref[0:1, :]


@jax.jit
def kernel(trajs, table):
    trajs = trajs.astype(jnp.int32)
    pe = jnp.broadcast_to(jnp.asarray(_PE_MEAN)[None, :], (8, _E))

    counts = _sc_hist(trajs).reshape(_SPLIT, _STRIDE)

    out_tc = pl.pallas_call(
        _tc_hist_mm_body,
        grid=(_BTC // _BR1,),
        in_specs=[
            pl.BlockSpec((_BR1, _L), lambda i: (i, 0)),
            pl.BlockSpec((_V, _E), lambda i: (0, 0)),
            pl.BlockSpec((8, _E), lambda i: (0, 0)),
        ],
        out_specs=pl.BlockSpec((_BR1, _E), lambda i: (i, 0)),
        out_shape=jax.ShapeDtypeStruct((_B, _E), jnp.float32),
    )(trajs, table, pe)

    tab_pad = jnp.zeros((_STRIDE, _E), jnp.float32).at[:_V].set(table)
    return pl.pallas_call(
        _mm_body,
        grid=(_SPLIT // _BR2,),
        in_specs=[
            pl.BlockSpec((_BR2, _STRIDE), lambda i: (i, 0)),
            pl.BlockSpec((_STRIDE, _E), lambda i: (0, 0)),
            pl.BlockSpec((8, _E), lambda i: (0, 0)),
            pl.BlockSpec(memory_space=pl.ANY),
        ],
        out_specs=pl.BlockSpec((_BR2, _E), lambda i: (i + _BTC // _BR2, 0)),
        out_shape=jax.ShapeDtypeStruct((_B, _E), jnp.float32),
        input_output_aliases={3: 0},
    )(counts, tab_pad, pe, out_tc)
